# baseline (speedup 1.0000x reference)
import functools

import jax
import jax.numpy as jnp
from jax import lax
from jax.experimental import pallas as pl
from jax.experimental.pallas import tpu as pltpu
from jax.experimental.pallas import tpu_sc as plsc

_NC = 2
_NS = 16
_NW = _NC * _NS

_RELAYOUT_BLOCK = 2048


def _relayout_body(src_ref, out_ref):
    out_ref[...] = jnp.transpose(src_ref[...], (1, 0))


@functools.lru_cache(maxsize=None)
def _make_relayout(E, V):
    grid = (V + _RELAYOUT_BLOCK - 1) // _RELAYOUT_BLOCK
    return pl.pallas_call(
        _relayout_body,
        grid=(grid,),
        in_specs=[pl.BlockSpec((E, _RELAYOUT_BLOCK), lambda i: (0, i))],
        out_specs=pl.BlockSpec((_RELAYOUT_BLOCK, E), lambda i: (i, 0)),
        out_shape=jax.ShapeDtypeStruct((V, E), jnp.float32),
    )


@functools.lru_cache(maxsize=None)
def _make_sc_gather(V, D, B):
    assert B % (16 * _NW) == 0 and D % 16 == 0
    b_per_w = B // _NW
    mesh = plsc.VectorSubcoreMesh(core_axis_name="c", subcore_axis_name="s")

    @functools.partial(
        pl.kernel,
        mesh=mesh,
        out_type=jax.ShapeDtypeStruct((B, D), jnp.float32),
        scratch_types=[
            pltpu.VMEM((b_per_w,), jnp.int32),
            pltpu.VMEM((b_per_w, D), jnp.float32),
            pltpu.SemaphoreType.DMA,
        ],
    )
    def gather_kernel(table_hbm, idx_hbm, out_hbm, idx_v, rows_v, sem):
        wid = lax.axis_index("s") * _NC + lax.axis_index("c")
        base = wid * b_per_w
        pltpu.sync_copy(idx_hbm.at[pl.ds(base, b_per_w)], idx_v)

        def issue(ci, carry):
            vec = idx_v[pl.ds(ci * 16, 16)]
            for l in range(16):
                r = vec[l]
                pltpu.async_copy(table_hbm.at[pl.ds(r, 1)],
                                 rows_v.at[pl.ds(ci * 16 + l, 1)], sem)
            return carry

        lax.fori_loop(0, b_per_w // 16, issue, 0)
        pltpu.make_async_copy(table_hbm.at[pl.ds(0, b_per_w)], rows_v,
                              sem).wait()
        pltpu.sync_copy(rows_v, out_hbm.at[pl.ds(base, b_per_w)])

    return gather_kernel


def _lstm_body(x_ref, wih_ref, whh_ref, b_ref, wfc_ref, bfc_ref,
               out_ref, h_scr, c_scr):
    T = x_ref.shape[0]
    H = whh_ref.shape[0]
    h_scr[...] = jnp.zeros_like(h_scr)
    c_scr[...] = jnp.zeros_like(c_scr)

    def step(t, carry):
        xt = x_ref[t].astype(jnp.bfloat16)
        gates = (
            jnp.dot(xt, wih_ref[...], preferred_element_type=jnp.float32)
            + jnp.dot(h_scr[...], whh_ref[...],
                      preferred_element_type=jnp.float32)
            + b_ref[...]
        )
        i = jax.nn.sigmoid(gates[:, :H])
        f = jax.nn.sigmoid(gates[:, H:2 * H])
        g = jnp.tanh(gates[:, 2 * H:3 * H])
        o = jax.nn.sigmoid(gates[:, 3 * H:])
        c = f * c_scr[...] + i * g
        c_scr[...] = c
        h_scr[...] = (o * jnp.tanh(c)).astype(jnp.bfloat16)
        return carry

    lax.fori_loop(0, T, step, 0)
    out_ref[...] = (
        jnp.dot(h_scr[...], wfc_ref[...], preferred_element_type=jnp.float32)
        + bfc_ref[...]
    )


def kernel(text, emb, W_ih, W_hh, b_ih, b_hh, W_fc, b_fc):
    T, B = text.shape
    V, E = emb.shape
    H = W_hh.shape[1]
    NC = W_fc.shape[0]

    table_bf = _make_relayout(E, V)(emb.T)
    idx = text.reshape(T * B)
    x_flat = _make_sc_gather(V, E, T * B)(table_bf, idx)
    x = x_flat.reshape(T, B, E)

    wih_t = W_ih.T.astype(jnp.bfloat16)
    whh_t = W_hh.T.astype(jnp.bfloat16)
    bias = (b_ih + b_hh).reshape(1, 4 * H)
    NCP = 128
    wfc_t = jnp.zeros((H, NCP), jnp.bfloat16).at[:, :NC].set(
        W_fc.T.astype(jnp.bfloat16))
    bfc = jnp.zeros((1, NCP), jnp.float32).at[:, :NC].set(b_fc)

    out = pl.pallas_call(
        _lstm_body,
        out_shape=jax.ShapeDtypeStruct((B, NCP), jnp.float32),
        scratch_shapes=[
            pltpu.VMEM((B, H), jnp.bfloat16),
            pltpu.VMEM((B, H), jnp.float32),
        ],
    )(x, wih_t, whh_t, bias, wfc_t, bfc)
    return out[:, :NC]


# speedup vs baseline: 1.0085x; 1.0085x over previous
import functools

import jax
import jax.numpy as jnp
from jax import lax
from jax.experimental import pallas as pl
from jax.experimental.pallas import tpu as pltpu
from jax.experimental.pallas import tpu_sc as plsc

_NC = 2
_NS = 16
_NW = _NC * _NS

_RELAYOUT_BLOCK = 2048


def _relayout_body(src_ref, out_ref):
    t = jnp.transpose(src_ref[...], (1, 0))
    h = t.shape[0] // 2
    out_ref[...] = jnp.concatenate([t[:h], t[h:]], axis=1)


@functools.lru_cache(maxsize=None)
def _make_relayout(E, V):
    grid = (V + _RELAYOUT_BLOCK - 1) // _RELAYOUT_BLOCK
    return pl.pallas_call(
        _relayout_body,
        grid=(grid,),
        in_specs=[pl.BlockSpec((E, _RELAYOUT_BLOCK), lambda i: (0, i))],
        out_specs=pl.BlockSpec((_RELAYOUT_BLOCK // 2, 2 * E), lambda i: (i, 0)),
        out_shape=jax.ShapeDtypeStruct((grid * (_RELAYOUT_BLOCK // 2), 2 * E),
                                       jnp.float32),
    )


@functools.lru_cache(maxsize=None)
def _make_sc_gather(V, D, B):
    assert B % (16 * _NW) == 0 and D % 16 == 0
    b_per_w = B // _NW
    mesh = plsc.VectorSubcoreMesh(core_axis_name="c", subcore_axis_name="s")

    @functools.partial(
        pl.kernel,
        mesh=mesh,
        out_type=jax.ShapeDtypeStruct((B, D), jnp.float32),
        scratch_types=[
            pltpu.VMEM((b_per_w,), jnp.int32),
            pltpu.VMEM((b_per_w, D), jnp.float32),
            pltpu.SemaphoreType.DMA,
        ],
    )
    def gather_kernel(table_hbm, idx_hbm, out_hbm, idx_v, rows_v, sem):
        wid = lax.axis_index("s") * _NC + lax.axis_index("c")
        base = wid * b_per_w
        pltpu.sync_copy(idx_hbm.at[pl.ds(base, b_per_w)], idx_v)
        pltpu.async_copy(table_hbm.at[idx_v], rows_v, sem).wait()
        pltpu.sync_copy(rows_v, out_hbm.at[pl.ds(base, b_per_w)])

    return gather_kernel


def _lstm_body(x_ref, par_ref, wih_ref, whh_ref, b_ref, wfc_ref, bfc_ref,
               out_ref, h_scr, c_scr):
    T = x_ref.shape[0]
    B = x_ref.shape[1]
    E2 = x_ref.shape[2]
    H = whh_ref.shape[0]
    h_scr[...] = jnp.zeros_like(h_scr)
    c_scr[...] = jnp.zeros_like(c_scr)
    col = lax.broadcasted_iota(jnp.int32, (B, E2), 1)

    def step(t, carry):
        p = par_ref[t]
        want_low = (col < E2 // 2) == (p == 0)
        xt = jnp.where(want_low, x_ref[t], 0.0).astype(jnp.bfloat16)
        gates = (
            jnp.dot(xt, wih_ref[...], preferred_element_type=jnp.float32)
            + jnp.dot(h_scr[...], whh_ref[...],
                      preferred_element_type=jnp.float32)
            + b_ref[...]
        )
        i = jax.nn.sigmoid(gates[:, :H])
        f = jax.nn.sigmoid(gates[:, H:2 * H])
        g = jnp.tanh(gates[:, 2 * H:3 * H])
        o = jax.nn.sigmoid(gates[:, 3 * H:])
        c = f * c_scr[...] + i * g
        c_scr[...] = c
        h_scr[...] = (o * jnp.tanh(c)).astype(jnp.bfloat16)
        return carry

    lax.fori_loop(0, T, step, 0)
    out_ref[...] = (
        jnp.dot(h_scr[...], wfc_ref[...], preferred_element_type=jnp.float32)
        + bfc_ref[...]
    )


def kernel(text, emb, W_ih, W_hh, b_ih, b_hh, W_fc, b_fc):
    T, B = text.shape
    V, E = emb.shape
    H = W_hh.shape[1]
    NC = W_fc.shape[0]

    table_p = _make_relayout(E, V)(emb.T)
    idx = text.reshape(T * B)
    idx2 = ((idx >> 11) << 10) | (idx & 1023)
    x_flat = _make_sc_gather(table_p.shape[0], 2 * E, T * B)(table_p, idx2)
    x = x_flat.reshape(T, B, 2 * E)
    par = ((text >> 10) & 1).reshape(T, B, 1)

    wih_t = jnp.concatenate([W_ih.T, W_ih.T], axis=0).astype(jnp.bfloat16)
    whh_t = W_hh.T.astype(jnp.bfloat16)
    bias = (b_ih + b_hh).reshape(1, 4 * H)
    NCP = 128
    wfc_t = jnp.zeros((H, NCP), jnp.bfloat16).at[:, :NC].set(
        W_fc.T.astype(jnp.bfloat16))
    bfc = jnp.zeros((1, NCP), jnp.float32).at[:, :NC].set(b_fc)

    out = pl.pallas_call(
        _lstm_body,
        out_shape=jax.ShapeDtypeStruct((B, NCP), jnp.float32),
        scratch_shapes=[
            pltpu.VMEM((B, H), jnp.bfloat16),
            pltpu.VMEM((B, H), jnp.float32),
        ],
    )(x, par, wih_t, whh_t, bias, wfc_t, bfc)
    return out[:, :NC]


# speedup vs baseline: 1.6994x; 1.6851x over previous
import functools

import jax
import jax.numpy as jnp
from jax import lax
from jax.experimental import pallas as pl
from jax.experimental.pallas import tpu as pltpu
from jax.experimental.pallas import tpu_sc as plsc

_NC = 2
_NS = 16
_NW = _NC * _NS

_RELAYOUT_BLOCK = 32768


def _relayout_body(src_ref, out_ref):
    t = jnp.transpose(src_ref[...], (1, 0))
    h = t.shape[0] // 2
    out_ref[...] = jnp.concatenate([t[:h], t[h:]], axis=1)


@functools.lru_cache(maxsize=None)
def _make_relayout(E, V):
    grid = (V + _RELAYOUT_BLOCK - 1) // _RELAYOUT_BLOCK
    return pl.pallas_call(
        _relayout_body,
        grid=(grid,),
        in_specs=[pl.BlockSpec((E, _RELAYOUT_BLOCK), lambda i: (0, i))],
        out_specs=pl.BlockSpec((_RELAYOUT_BLOCK // 2, 2 * E), lambda i: (i, 0)),
        out_shape=jax.ShapeDtypeStruct((grid * (_RELAYOUT_BLOCK // 2), 2 * E),
                                       jnp.float32),
        compiler_params=pltpu.CompilerParams(
            vmem_limit_bytes=128 * 1024 * 1024),
    )


@functools.lru_cache(maxsize=None)
def _make_sc_gather(V, D, B):
    assert B % (16 * _NW) == 0 and D % 16 == 0
    b_per_w = B // _NW
    mesh = plsc.VectorSubcoreMesh(core_axis_name="c", subcore_axis_name="s")

    @functools.partial(
        pl.kernel,
        mesh=mesh,
        out_type=jax.ShapeDtypeStruct((B, D), jnp.float32),
        scratch_types=[
            pltpu.VMEM((b_per_w,), jnp.int32),
            pltpu.VMEM((b_per_w, D), jnp.float32),
            pltpu.SemaphoreType.DMA,
        ],
    )
    def gather_kernel(table_hbm, idx_hbm, out_hbm, idx_v, rows_v, sem):
        wid = lax.axis_index("s") * _NC + lax.axis_index("c")
        base = wid * b_per_w
        pltpu.sync_copy(idx_hbm.at[pl.ds(base, b_per_w)], idx_v)
        pltpu.async_copy(table_hbm.at[idx_v], rows_v, sem).wait()
        pltpu.sync_copy(rows_v, out_hbm.at[pl.ds(base, b_per_w)])

    return gather_kernel


def _lstm_body(x_ref, par_ref, wih_ref, whh_ref, b_ref, wfc_ref, bfc_ref,
               out_ref, h_scr, c_scr):
    E2 = x_ref.shape[1]
    H = whh_ref.shape[0]
    B = h_scr.shape[0]
    T = x_ref.shape[0] // B
    h_scr[...] = jnp.zeros_like(h_scr)
    c_scr[...] = jnp.zeros_like(c_scr)
    col = lax.broadcasted_iota(jnp.int32, (B, E2), 1)

    def step(t, carry):
        p = par_ref[t]
        want_low = (col < E2 // 2) == (p == 0)
        xt = jnp.where(want_low, x_ref[pl.ds(t * B, B)],
                       0.0).astype(jnp.bfloat16)
        gates = (
            jnp.dot(xt, wih_ref[...], preferred_element_type=jnp.float32)
            + jnp.dot(h_scr[...], whh_ref[...],
                      preferred_element_type=jnp.float32)
            + b_ref[...]
        )
        i = jax.nn.sigmoid(gates[:, :H])
        f = jax.nn.sigmoid(gates[:, H:2 * H])
        g = jnp.tanh(gates[:, 2 * H:3 * H])
        o = jax.nn.sigmoid(gates[:, 3 * H:])
        c = f * c_scr[...] + i * g
        c_scr[...] = c
        h_scr[...] = (o * jnp.tanh(c)).astype(jnp.bfloat16)
        return carry

    lax.fori_loop(0, T, step, 0)
    out_ref[...] = (
        jnp.dot(h_scr[...], wfc_ref[...], preferred_element_type=jnp.float32)
        + bfc_ref[...]
    )


def kernel(text, emb, W_ih, W_hh, b_ih, b_hh, W_fc, b_fc):
    T, B = text.shape
    V, E = emb.shape
    H = W_hh.shape[1]
    NC = W_fc.shape[0]

    table_p = _make_relayout(E, V)(emb.T)
    idx = text.reshape(T * B)
    lb = _RELAYOUT_BLOCK.bit_length() - 1
    hb = lb - 1
    idx2 = ((idx >> lb) << hb) | (idx & ((1 << hb) - 1))
    x_flat = _make_sc_gather(table_p.shape[0], 2 * E, T * B)(table_p, idx2)
    par = ((text >> hb) & 1).reshape(T, B, 1)

    wih_t = jnp.concatenate([W_ih.T, W_ih.T], axis=0).astype(jnp.bfloat16)
    whh_t = W_hh.T.astype(jnp.bfloat16)
    bias = (b_ih + b_hh).reshape(1, 4 * H)
    NCP = 128
    wfc_t = jnp.zeros((H, NCP), jnp.bfloat16).at[:, :NC].set(
        W_fc.T.astype(jnp.bfloat16))
    bfc = jnp.zeros((1, NCP), jnp.float32).at[:, :NC].set(b_fc)

    out = pl.pallas_call(
        _lstm_body,
        out_shape=jax.ShapeDtypeStruct((B, NCP), jnp.float32),
        scratch_shapes=[
            pltpu.VMEM((B, H), jnp.bfloat16),
            pltpu.VMEM((B, H), jnp.float32),
        ],
    )(x_flat, par, wih_t, whh_t, bias, wfc_t, bfc)
    return out[:, :NC]
